# baseline (speedup 1.0000x reference)
import jax
import jax.numpy as jnp
from jax.experimental import pallas as pl
from jax.experimental.pallas import tpu as pltpu

_N = 14
_T = 8


def _make_kernel(t):
    def _gcn_fused_kernel(x_ref, a_ref, w_ref, b_ref, fw_ref, fb_ref, o_ref):
        j = pl.program_id(0)
        n = a_ref.shape[0]

        a = a_ref[...]
        row = jax.lax.broadcasted_iota(jnp.int32, (n, n), 0)
        col = jax.lax.broadcasted_iota(jnp.int32, (n, n), 1)
        eye = (row == col).astype(jnp.float32)
        a = jnp.where(a_ref[0:1, 0:1] == 0.0, a + eye, a)

        xw = jnp.dot(x_ref[...], w_ref[...],
                     preferred_element_type=jnp.float32)
        axw = jnp.dot(a, xw, preferred_element_type=jnp.float32)
        gc = jnp.maximum(axw, 0.0) + b_ref[...]

        part = jnp.sum(fw_ref[...] * gc[None, :, :],
                       axis=(1, 2)).reshape(1, n)

        @pl.when(j == 0)
        def _init():
            o_ref[...] = part + fb_ref[...]

        @pl.when(j > 0)
        def _acc():
            o_ref[...] += part

    return _gcn_fused_kernel


def kernel(x, adj, gc_weight, gc_bias, fc_weight, fc_bias):
    n = _N
    x2d = x.reshape(n, -1).astype(jnp.float32)
    f_dim = x2d.shape[1]
    p_dim = gc_weight.shape[1]
    w = gc_weight.astype(jnp.float32)
    a = adj.astype(jnp.float32)
    b2 = gc_bias.reshape(1, p_dim).astype(jnp.float32)
    fw3 = fc_weight.reshape(n, n, p_dim).astype(jnp.float32)
    fb2 = fc_bias.reshape(1, n).astype(jnp.float32)

    t = _T if p_dim % (_T * 128) == 0 else 1
    tp = p_dim // t

    return pl.pallas_call(
        _make_kernel(t),
        grid=(t,),
        in_specs=[
            pl.BlockSpec((n, f_dim), lambda j: (0, 0)),
            pl.BlockSpec((n, n), lambda j: (0, 0)),
            pl.BlockSpec((f_dim, tp), lambda j: (0, j)),
            pl.BlockSpec((1, tp), lambda j: (0, j)),
            pl.BlockSpec((n, n, tp), lambda j: (0, 0, j)),
            pl.BlockSpec((1, n), lambda j: (0, 0)),
        ],
        out_specs=pl.BlockSpec((1, n), lambda j: (0, 0)),
        out_shape=jax.ShapeDtypeStruct((1, n), jnp.float32),
        compiler_params=pltpu.CompilerParams(
            dimension_semantics=("arbitrary",)),
    )(x2d, a, w, b2, fw3, fb2)


# speedup vs baseline: 1.0402x; 1.0402x over previous
import jax
import jax.numpy as jnp
from jax.experimental import pallas as pl
from jax.experimental.pallas import tpu as pltpu

_N = 14
_T = 2


def _make_kernel(t):
    def _gcn_fused_kernel(x_ref, a_ref, w_ref, b_ref, fw_ref, fb_ref, o_ref):
        j = pl.program_id(0)
        n = a_ref.shape[0]

        a = a_ref[...]
        row = jax.lax.broadcasted_iota(jnp.int32, (n, n), 0)
        col = jax.lax.broadcasted_iota(jnp.int32, (n, n), 1)
        eye = (row == col).astype(jnp.float32)
        a = jnp.where(a_ref[0:1, 0:1] == 0.0, a + eye, a)

        xw = jnp.dot(x_ref[...], w_ref[...],
                     preferred_element_type=jnp.float32)
        axw = jnp.dot(a, xw, preferred_element_type=jnp.float32)
        gc = jnp.maximum(axw, 0.0) + b_ref[...]

        part = jnp.sum(fw_ref[...] * gc[None, :, :],
                       axis=(1, 2)).reshape(1, n)

        @pl.when(j == 0)
        def _init():
            o_ref[...] = part + fb_ref[...]

        @pl.when(j > 0)
        def _acc():
            o_ref[...] += part

    return _gcn_fused_kernel


def kernel(x, adj, gc_weight, gc_bias, fc_weight, fc_bias):
    n = _N
    x2d = x.reshape(n, -1).astype(jnp.float32)
    f_dim = x2d.shape[1]
    p_dim = gc_weight.shape[1]
    w = gc_weight.astype(jnp.float32)
    a = adj.astype(jnp.float32)
    b2 = gc_bias.reshape(1, p_dim).astype(jnp.float32)
    fw3 = fc_weight.reshape(n, n, p_dim).astype(jnp.float32)
    fb2 = fc_bias.reshape(1, n).astype(jnp.float32)

    t = _T if p_dim % (_T * 128) == 0 else 1
    tp = p_dim // t

    return pl.pallas_call(
        _make_kernel(t),
        grid=(t,),
        in_specs=[
            pl.BlockSpec((n, f_dim), lambda j: (0, 0)),
            pl.BlockSpec((n, n), lambda j: (0, 0)),
            pl.BlockSpec((f_dim, tp), lambda j: (0, j)),
            pl.BlockSpec((1, tp), lambda j: (0, j)),
            pl.BlockSpec((n, n, tp), lambda j: (0, 0, j)),
            pl.BlockSpec((1, n), lambda j: (0, 0)),
        ],
        out_specs=pl.BlockSpec((1, n), lambda j: (0, 0)),
        out_shape=jax.ShapeDtypeStruct((1, n), jnp.float32),
        compiler_params=pltpu.CompilerParams(
            dimension_semantics=("arbitrary",)),
    )(x2d, a, w, b2, fw3, fb2)
